# baseline (speedup 1.0000x reference)
import functools

import jax
import jax.numpy as jnp
from jax import lax
from jax.experimental import pallas as pl
from jax.experimental.pallas import tpu as pltpu
from jax.experimental.pallas import tpu_sc as plsc

_LANES = 16
_CHUNK = 128


def _make_kernel(B, D, NC, NS, V2):
    NW = NC * NS
    BW = B // NW
    NCHUNK = BW // _CHUNK
    NGRP = _CHUNK // _LANES
    mesh = plsc.VectorSubcoreMesh(core_axis_name="c", subcore_axis_name="s")

    @functools.partial(
        pl.kernel,
        mesh=mesh,
        out_type=jax.ShapeDtypeStruct((B,), jnp.float32),
        compiler_params=pltpu.CompilerParams(needs_layout_passes=False),
        scratch_types=[
            pltpu.VMEM((BW,), jnp.int32),
            pltpu.VMEM((BW,), jnp.int32),
            pltpu.VMEM((2, _CHUNK, 64), jnp.float32),
            pltpu.VMEM((2, _CHUNK, 64), jnp.float32),
            pltpu.VMEM((BW,), jnp.float32),
            pltpu.SemaphoreType.DMA,
            pltpu.SemaphoreType.DMA,
        ],
    )
    def k(user_hbm, item_hbm, uemb_hbm, iemb_hbm, out_hbm,
          usm, ism, ubuf, ibuf, outv, sem0, sem1):
        wid = lax.axis_index("s") * NC + lax.axis_index("c")
        base = wid * BW
        sems = (sem0, sem1)

        pltpu.sync_copy(user_hbm.at[pl.ds(base, BW)], usm)
        pltpu.sync_copy(item_hbm.at[pl.ds(base, BW)], ism)

        def issue(chunk, p):
            def row16(g, carry):
                uvec = usm[pl.ds(chunk * _CHUNK + g * _LANES, _LANES)] >> 1
                ivec = ism[pl.ds(chunk * _CHUNK + g * _LANES, _LANES)] >> 1
                for r in range(_LANES):
                    lr = g * _LANES + r
                    pltpu.async_copy(
                        uemb_hbm.at[pl.ds(uvec[r], 1)],
                        ubuf.at[p].at[pl.ds(lr, 1)], sems[p])
                    pltpu.async_copy(
                        iemb_hbm.at[pl.ds(ivec[r], 1)],
                        ibuf.at[p].at[pl.ds(lr, 1)], sems[p])
                return carry
            lax.fori_loop(0, NGRP, row16, 0)

        def drain(p):
            pltpu.make_async_copy(
                uemb_hbm.at[pl.ds(0, _CHUNK)], ubuf.at[p], sems[p]).wait()
            pltpu.make_async_copy(
                iemb_hbm.at[pl.ds(0, _CHUNK)], ibuf.at[p], sems[p]).wait()

        lane = lax.iota(jnp.int32, _LANES)
        issue(0, 0)
        for chunk in range(NCHUNK):
            p = chunk & 1
            if chunk + 1 < NCHUNK:
                issue(chunk + 1, 1 - p)
            drain(p)

            def grp(g, carry, chunk=chunk, p=p):
                uvec = usm[pl.ds(chunk * _CHUNK + g * _LANES, _LANES)]
                ivec = ism[pl.ds(chunk * _CHUNK + g * _LANES, _LANES)]
                acc = jnp.zeros((_LANES,), jnp.float32)
                for r in range(_LANES):
                    lr = g * _LANES + r
                    pu = (uvec[r] & 1) == 0
                    pi = (ivec[r] & 1) == 0
                    s = jnp.zeros((_LANES,), jnp.float32)
                    for c in range(D // _LANES):
                        sl = pl.ds(c * _LANES, _LANES)
                        ua, ub = plsc.unpack(
                            plsc.bitcast(ubuf[p, lr, sl], jnp.bfloat16),
                            format=plsc.PackFormat.INTERLEAVED)
                        va, vb = plsc.unpack(
                            plsc.bitcast(ibuf[p, lr, sl], jnp.bfloat16),
                            format=plsc.PackFormat.INTERLEAVED)
                        uval = jnp.where(pu, ua, ub)
                        vval = jnp.where(pi, va, vb)
                        s = s + uval * vval
                    acc = jnp.where(lane == r, jnp.sum(s), acc)
                outv[pl.ds(chunk * _CHUNK + g * _LANES, _LANES)] = acc
                return carry

            lax.fori_loop(0, NGRP, grp, 0)

        pltpu.sync_copy(outv, out_hbm.at[pl.ds(base, BW)])

    return k


def _bf16_bits(xu):
    lsb = (xu >> 16) & jnp.uint32(1)
    return (xu + jnp.uint32(0x7FFF) + lsb) >> 16


_TBLK = 12800


def _tpose_pack(embT):
    D, V = embT.shape
    nblk = (V + _TBLK - 1) // _TBLK

    def body(x_ref, o_ref):
        t = x_ref[...].T.reshape(_TBLK // 2, 2, D)
        lo = jax.lax.bitcast_convert_type(t[:, 0, :], jnp.uint32)
        hi = jax.lax.bitcast_convert_type(t[:, 1, :], jnp.uint32)
        o_ref[...] = jax.lax.bitcast_convert_type(
            _bf16_bits(lo) | (_bf16_bits(hi) << 16), jnp.float32)

    return pl.pallas_call(
        body,
        grid=(nblk,),
        in_specs=[pl.BlockSpec((D, _TBLK), lambda i: (0, i))],
        out_specs=pl.BlockSpec((_TBLK // 2, D), lambda i: (i, 0)),
        out_shape=jax.ShapeDtypeStruct((V // 2, D), jnp.float32),
    )(embT)


@jax.jit
def kernel(user, item, user_emb, item_emb):
    B = user.shape[0]
    D = user_emb.shape[1]
    info = plsc.get_sparse_core_info()
    k = _make_kernel(B, D, info.num_cores, info.num_subcores,
                     user_emb.shape[0] // 2)
    return k(user.astype(jnp.int32), item.astype(jnp.int32),
             _tpose_pack(user_emb.T), _tpose_pack(item_emb.T))


# speedup vs baseline: 2.2219x; 2.2219x over previous
import functools

import jax
import jax.numpy as jnp
from jax import lax
from jax.experimental import pallas as pl
from jax.experimental.pallas import tpu as pltpu
from jax.experimental.pallas import tpu_sc as plsc

_LANES = 16
_CHUNK = 128


def _make_kernel(B, D, NC, NS):
    NW = NC * NS
    BW = B // NW
    NCHUNK = BW // _CHUNK
    NGRP = _CHUNK // _LANES
    mesh = plsc.VectorSubcoreMesh(core_axis_name="c", subcore_axis_name="s")

    @functools.partial(
        pl.kernel,
        mesh=mesh,
        out_type=jax.ShapeDtypeStruct((B,), jnp.float32),
        compiler_params=pltpu.CompilerParams(needs_layout_passes=False),
        scratch_types=[
            pltpu.VMEM((BW,), jnp.int32),
            pltpu.VMEM((BW,), jnp.int32),
            pltpu.VMEM((2, _CHUNK, 64), jnp.float32),
            pltpu.VMEM((2, _CHUNK, 64), jnp.float32),
            pltpu.VMEM((BW,), jnp.float32),
            pltpu.SemaphoreType.DMA,
            pltpu.SemaphoreType.DMA,
        ],
    )
    def k(user_hbm, item_hbm, uemb_hbm, iemb_hbm, out_hbm,
          usm, ism, ubuf, ibuf, outv, sem0, sem1):
        wid = lax.axis_index("s") * NC + lax.axis_index("c")
        base = wid * BW
        sems = (sem0, sem1)

        pltpu.sync_copy(user_hbm.at[pl.ds(base, BW)], usm)
        pltpu.sync_copy(item_hbm.at[pl.ds(base, BW)], ism)

        def issue(chunk, p):
            def row16(g, carry):
                uvec = usm[pl.ds(chunk * _CHUNK + g * _LANES, _LANES)]
                ivec = ism[pl.ds(chunk * _CHUNK + g * _LANES, _LANES)]
                for r in range(_LANES):
                    lr = g * _LANES + r
                    pltpu.async_copy(
                        uemb_hbm.at[pl.ds(uvec[r], 1)],
                        ubuf.at[p].at[pl.ds(lr, 1)], sems[p])
                    pltpu.async_copy(
                        iemb_hbm.at[pl.ds(ivec[r], 1)],
                        ibuf.at[p].at[pl.ds(lr, 1)], sems[p])
                return carry
            lax.fori_loop(0, NGRP, row16, 0)

        def drain(p):
            pltpu.make_async_copy(
                uemb_hbm.at[pl.ds(0, _CHUNK)], ubuf.at[p], sems[p]).wait()
            pltpu.make_async_copy(
                iemb_hbm.at[pl.ds(0, _CHUNK)], ibuf.at[p], sems[p]).wait()

        lane = lax.iota(jnp.int32, _LANES)
        issue(0, 0)
        for chunk in range(NCHUNK):
            p = chunk & 1
            if chunk + 1 < NCHUNK:
                issue(chunk + 1, 1 - p)
            drain(p)

            def grp(g, carry, chunk=chunk, p=p):
                acc = jnp.zeros((_LANES,), jnp.float32)
                for r in range(_LANES):
                    lr = g * _LANES + r
                    s = jnp.zeros((_LANES,), jnp.float32)
                    for c in range(D // _LANES):
                        sl = pl.ds(c * _LANES, _LANES)
                        s = s + ubuf[p, lr, sl] * ibuf[p, lr, sl]
                    acc = jnp.where(lane == r, jnp.sum(s), acc)
                outv[pl.ds(chunk * _CHUNK + g * _LANES, _LANES)] = acc
                return carry

            lax.fori_loop(0, NGRP, grp, 0)

        pltpu.sync_copy(outv, out_hbm.at[pl.ds(base, BW)])

    return k


_TBLK = 12800


def _tpose(embT):
    D, V = embT.shape
    nblk = (V + _TBLK - 1) // _TBLK

    def body(x_ref, o_ref):
        o_ref[...] = x_ref[...].T

    return pl.pallas_call(
        body,
        grid=(nblk,),
        in_specs=[pl.BlockSpec((D, _TBLK), lambda i: (0, i))],
        out_specs=pl.BlockSpec((_TBLK, D), lambda i: (i, 0)),
        out_shape=jax.ShapeDtypeStruct((V, D), jnp.float32),
    )(embT)


@jax.jit
def kernel(user, item, user_emb, item_emb):
    B = user.shape[0]
    D = user_emb.shape[1]
    info = plsc.get_sparse_core_info()
    k = _make_kernel(B, D, info.num_cores, info.num_subcores)
    return k(user.astype(jnp.int32), item.astype(jnp.int32),
             _tpose(user_emb.T), _tpose(item_emb.T))
